# baseline (speedup 1.0000x reference)
import functools

import jax
import jax.numpy as jnp
from jax import lax
from jax.experimental import pallas as pl
from jax.experimental.pallas import tpu as pltpu
from jax.experimental.pallas import tpu_sc as plsc
from jax._src.pallas.einshape import einshape as _einshape

NC = 2
NS = 16
NW = NC * NS

BATCH = 4096
HIST = 200
EMBED = 32
VOCAB = 1000000
PACK = 4
PACKED_W = EMBED * PACK

TBLK = 4096
TGRID = (VOCAB + TBLK - 1) // TBLK
PACKED_ROWS = TGRID * (TBLK // PACK)

ROWS_PER_W = BATCH // NW
CHUNK = 100
CHUNKS_PER_W = ROWS_PER_W * 2
NBUF = 2


def _pack_body(in_ref, out_ref):
    x = in_ref[...]
    eye = jnp.eye(EMBED, dtype=jnp.float32)
    for a in range(PACK):
        sl = x[:, a * (TBLK // PACK):(a + 1) * (TBLK // PACK)]
        t = jax.lax.dot_general(
            sl, eye, (((0,), (0,)), ((), ())),
            precision=jax.lax.Precision.HIGHEST,
        )
        out_ref[:, a * EMBED:(a + 1) * EMBED] = t


@jax.jit
def _tc_pack(tab_t):
    return pl.pallas_call(
        _pack_body,
        grid=(TGRID,),
        in_specs=[pl.BlockSpec((EMBED, TBLK), lambda g: (0, g))],
        out_specs=pl.BlockSpec((TBLK // PACK, PACKED_W), lambda g: (g, 0)),
        out_shape=jax.ShapeDtypeStruct((PACKED_ROWS, PACKED_W), jnp.float32),
    )(tab_t)


def _body(p_hbm, off_hbm, table_hbm, out_hbm, p_v, off_v, bufs, out_v,
          s0, s1):
    sems = (s0, s1)
    wid = lax.axis_index("s") * NC + lax.axis_index("c")

    pltpu.sync_copy(p_hbm.at[wid], p_v)
    pltpu.sync_copy(off_hbm.at[wid], off_v)

    for i in range(NBUF):
        pltpu.async_copy(table_hbm.at[p_v.at[i]], bufs.at[i], sems[i])

    def sum_chunk(i, c, acc0, acc1):
        for g in range(0, CHUNK, 16):
            ov = off_v[c, pl.ds(g, 16)]
            for l in range(min(16, CHUNK - g)):
                o = ov[l]
                r = g + l
                acc0 = acc0 + bufs[i, r, pl.ds(o, 16)]
                acc1 = acc1 + bufs[i, r, pl.ds(o + 16, 16)]
        return acc0, acc1

    zero = jnp.zeros((16,), jnp.float32)

    def j_body(j, carry):
        acc0, acc1 = zero, zero
        for i in range(NBUF):
            pltpu.make_async_copy(
                table_hbm.at[p_v.at[i]], bufs.at[i], sems[i]
            ).wait()
            acc0, acc1 = sum_chunk(i, NBUF * j + i, acc0, acc1)

            @pl.when(j < ROWS_PER_W - 1)
            def _():
                nxt = NBUF * j + NBUF + i
                pltpu.async_copy(
                    table_hbm.at[p_v.at[nxt]], bufs.at[i], sems[i]
                )

        out_v[j, 0:16] = acc0
        out_v[j, 16:32] = acc1
        return carry

    lax.fori_loop(0, ROWS_PER_W, j_body, 0)

    pltpu.sync_copy(out_v, out_hbm.at[wid])


@jax.jit
def _embed_sum(p, off, table2):
    mesh = plsc.VectorSubcoreMesh(
        core_axis_name="c", subcore_axis_name="s", num_cores=NC, num_subcores=NS
    )
    f = functools.partial(
        pl.kernel,
        mesh=mesh,
        out_type=jax.ShapeDtypeStruct((NW, ROWS_PER_W, EMBED), jnp.float32),
        scratch_types=[
            pltpu.VMEM((CHUNKS_PER_W, CHUNK), jnp.int32),
            pltpu.VMEM((CHUNKS_PER_W, PACKED_W), jnp.int32),
            pltpu.VMEM((NBUF, CHUNK, PACKED_W), jnp.float32),
            pltpu.VMEM((ROWS_PER_W, EMBED), jnp.float32),
            pltpu.SemaphoreType.DMA,
            pltpu.SemaphoreType.DMA,
        ],
        compiler_params=pltpu.CompilerParams(use_tc_tiling_on_sc=True),
    )(_body)
    return f(p, off, table2)


def kernel(ancestor_idx, embed_weight):
    idx = ancestor_idx.astype(jnp.int32)
    p = (((idx >> 12) << 10) | (idx & 1023)).reshape(NW, CHUNKS_PER_W, CHUNK)
    off = (((idx >> 10) & 3) << 5).reshape(NW, CHUNKS_PER_W, CHUNK)
    off = jnp.pad(off, ((0, 0), (0, 0), (0, PACKED_W - CHUNK)))
    table2 = _tc_pack(embed_weight.T)
    out = _embed_sum(p, off, table2)
    return out.reshape(BATCH, EMBED)


# speedup vs baseline: 1.5817x; 1.5817x over previous
import functools

import jax
import jax.numpy as jnp
from jax import lax
from jax.experimental import pallas as pl
from jax.experimental.pallas import tpu as pltpu
from jax.experimental.pallas import tpu_sc as plsc
from jax._src.pallas.einshape import einshape as _einshape

NC = 2
NS = 16
NW = NC * NS

BATCH = 4096
HIST = 200
EMBED = 32
VOCAB = 1000000
PACK = 4
PACKED_W = EMBED * PACK

TBLK = 8192
TGRID = (VOCAB + TBLK - 1) // TBLK
PACKED_ROWS = TGRID * (TBLK // PACK)

ROWS_PER_W = BATCH // NW
CHUNK = 100
CHUNKS_PER_W = ROWS_PER_W * 2
NBUF = 2


def _pack_body(in_ref, out_ref):
    x = in_ref[...]
    for a in range(PACK):
        sl = x[:, a * (TBLK // PACK):(a + 1) * (TBLK // PACK)]
        out_ref[:, a * EMBED:(a + 1) * EMBED] = jnp.transpose(sl, (1, 0))


@jax.jit
def _tc_pack(tab_t):
    return pl.pallas_call(
        _pack_body,
        grid=(TGRID,),
        in_specs=[pl.BlockSpec((EMBED, TBLK), lambda g: (0, g))],
        out_specs=pl.BlockSpec((TBLK // PACK, PACKED_W), lambda g: (g, 0)),
        out_shape=jax.ShapeDtypeStruct((PACKED_ROWS, PACKED_W), jnp.float32),
    )(tab_t)


def _body(p_hbm, off_hbm, table_hbm, out_hbm, p_v, off_v, bufs, out_v,
          s0, s1):
    sems = (s0, s1)
    wid = lax.axis_index("s") * NC + lax.axis_index("c")

    pltpu.sync_copy(p_hbm.at[wid], p_v)
    pltpu.sync_copy(off_hbm.at[wid], off_v)

    for i in range(NBUF):
        pltpu.async_copy(table_hbm.at[p_v.at[i]], bufs.at[i], sems[i])

    def sum_chunk(i, c, acc0, acc1):
        for g in range(0, CHUNK, 16):
            ov = off_v[c, pl.ds(g, 16)]
            for l in range(min(16, CHUNK - g)):
                o = ov[l]
                r = g + l
                acc0 = acc0 + bufs[i, r, pl.ds(o, 16)]
                acc1 = acc1 + bufs[i, r, pl.ds(o + 16, 16)]
        return acc0, acc1

    zero = jnp.zeros((16,), jnp.float32)

    def j_body(j, carry):
        acc0, acc1 = zero, zero
        for i in range(NBUF):
            pltpu.make_async_copy(
                table_hbm.at[p_v.at[i]], bufs.at[i], sems[i]
            ).wait()
            acc0, acc1 = sum_chunk(i, NBUF * j + i, acc0, acc1)

            @pl.when(j < ROWS_PER_W - 1)
            def _():
                nxt = NBUF * j + NBUF + i
                pltpu.async_copy(
                    table_hbm.at[p_v.at[nxt]], bufs.at[i], sems[i]
                )

        out_v[j, 0:16] = acc0
        out_v[j, 16:32] = acc1
        return carry

    lax.fori_loop(0, ROWS_PER_W, j_body, 0)

    pltpu.sync_copy(out_v, out_hbm.at[wid])


@jax.jit
def _embed_sum(p, off, table2):
    mesh = plsc.VectorSubcoreMesh(
        core_axis_name="c", subcore_axis_name="s", num_cores=NC, num_subcores=NS
    )
    f = functools.partial(
        pl.kernel,
        mesh=mesh,
        out_type=jax.ShapeDtypeStruct((NW, ROWS_PER_W, EMBED), jnp.float32),
        scratch_types=[
            pltpu.VMEM((CHUNKS_PER_W, CHUNK), jnp.int32),
            pltpu.VMEM((CHUNKS_PER_W, PACKED_W), jnp.int32),
            pltpu.VMEM((NBUF, CHUNK, PACKED_W), jnp.float32),
            pltpu.VMEM((ROWS_PER_W, EMBED), jnp.float32),
            pltpu.SemaphoreType.DMA,
            pltpu.SemaphoreType.DMA,
        ],
        compiler_params=pltpu.CompilerParams(use_tc_tiling_on_sc=True),
    )(_body)
    return f(p, off, table2)


def kernel(ancestor_idx, embed_weight):
    idx = ancestor_idx.astype(jnp.int32)
    q = TBLK // PACK
    p = ((idx // TBLK) * q + idx % q).reshape(NW, CHUNKS_PER_W, CHUNK)
    off = (((idx // q) % PACK) << 5).reshape(NW, CHUNKS_PER_W, CHUNK)
    off = jnp.pad(off, ((0, 0), (0, 0), (0, PACKED_W - CHUNK)))
    table2 = _tc_pack(embed_weight.T)
    out = _embed_sum(p, off, table2)
    return out.reshape(BATCH, EMBED)
